# baseline (speedup 1.0000x reference)
import functools

import jax
import jax.numpy as jnp
from jax import lax
from jax.experimental import pallas as pl
from jax.experimental.pallas import tpu as pltpu
from jax.experimental.pallas import tpu_sc as plsc

NP = 10240
K = 128
CHT = 160
TOTCH = 16 * CHT
EP = TOTCH * K
RPT = NP // 16
BT = 1024

_SC_PARAMS = pltpu.CompilerParams(needs_layout_passes=False,
                                  use_tc_tiling_on_sc=False)


def _sc_mesh():
    return plsc.VectorSubcoreMesh(core_axis_name="c", subcore_axis_name="s")


@functools.lru_cache(maxsize=None)
def _sc_edge_kernel(Hp2):
    nblk = Hp2 // 16

    @functools.partial(
        pl.kernel,
        out_type=jax.ShapeDtypeStruct((2, NP, Hp2), jnp.float32),
        mesh=_sc_mesh(),
        scratch_types=[
            pltpu.VMEM((8, 3, K), jnp.int32),
            pltpu.VMEM((4, K, Hp2), jnp.float32),
            pltpu.VMEM((64, Hp2), jnp.float32),
            pltpu.VMEM_SHARED((NP, Hp2), jnp.float32),
            pltpu.VMEM_SHARED((NP, Hp2), jnp.float32),
        ] + [pltpu.SemaphoreType.DMA] * 17,
        compiler_params=_SC_PARAMS,
    )
    def k(m_hbm, ed_hbm, agg_hbm, ebuf, rowsb, zbuf, agg_sh, m_sh, *sems):
        gsems = sems[0:4]
        ssems = sems[4:8]
        esems = sems[8:16]
        cid = lax.axis_index("c")
        sid = lax.axis_index("s")

        zeros16 = jnp.zeros((16,), jnp.float32)

        def zrow(i, carry):
            for j in range(nblk):
                zbuf[i, pl.ds(j * 16, 16)] = zeros16
            return carry

        lax.fori_loop(0, 64, zrow, 0)
        pltpu.async_copy(m_hbm.at[cid, pl.ds(sid * RPT, RPT)],
                         m_sh.at[pl.ds(sid * RPT, RPT)], sems[16])
        for r in range(RPT // 64):
            pltpu.async_copy(zbuf, agg_sh.at[pl.ds(sid * RPT + r * 64, 64)],
                             sems[16])
        pltpu.make_async_copy(m_hbm.at[cid, pl.ds(sid * RPT, RPT)],
                              m_sh.at[pl.ds(sid * RPT, RPT)],
                              sems[16]).wait()
        for r in range(RPT // 64):
            pltpu.make_async_copy(
                zbuf, agg_sh.at[pl.ds(sid * RPT + r * 64, 64)],
                sems[16]).wait()
        plsc.subcore_barrier()

        def edata_start(t, s):
            pltpu.async_copy(ed_hbm.at[sid * CHT + t], ebuf.at[s], esems[s])

        def edata_wait(t, s):
            pltpu.make_async_copy(ed_hbm.at[sid * CHT + t], ebuf.at[s],
                                  esems[s]).wait()

        def _m_src(s, hbm):
            if hbm:
                return m_hbm.at[cid].at[ebuf.at[s, 0]]
            return m_sh.at[ebuf.at[s, 0]]

        def gather_start(t, s, r, hbm=False):
            pltpu.async_copy(_m_src(s, hbm), rowsb.at[r], gsems[r])

        def gather_wait(t, s, r, hbm=False):
            pltpu.make_async_copy(_m_src(s, hbm), rowsb.at[r],
                                  gsems[r]).wait()

        def scatter_start(t, s, r):
            pltpu.async_copy(rowsb.at[r], agg_sh.at[ebuf.at[s, 1]], ssems[r],
                             add=True)

        def scatter_wait(t, s, r):
            pltpu.make_async_copy(rowsb.at[r], agg_sh.at[ebuf.at[s, 1]],
                                  ssems[r]).wait()

        def scale(s, r):
            s16 = jnp.full((16,), s, jnp.int32)
            two16 = jnp.full((16,), 2, jnp.int32)

            def ebody(g, carry):
                e0 = g * 4
                ews = []
                for u in range(4):
                    ewbits = plsc.load_gather(
                        ebuf, [s16, two16, jnp.full((16,), e0 + u,
                                                    jnp.int32)])
                    ews.append(plsc.bitcast(ewbits, jnp.float32))
                for u in range(4):
                    for j in range(nblk):
                        rowsb[r, e0 + u, pl.ds(j * 16, 16)] = (
                            rowsb[r, e0 + u, pl.ds(j * 16, 16)] * ews[u])
                return carry
            lax.fori_loop(0, K // 4, ebody, 0)

        for s in range(6):
            edata_start(s, s)
        edata_wait(0, 0)
        gather_start(0, 0, 0, hbm=False)
        edata_wait(1, 1)
        gather_start(1, 1, 1, hbm=True)

        def step(i, carry):
            for b in range(8):
                t = 8 * i + b
                gather_wait(t, b, b % 4, hbm=(b % 2 == 1))
                scale(b, b % 4)
                scatter_start(t, b, b % 4)

                @pl.when(t >= 2)
                def _():
                    scatter_wait(t - 2, (b - 2) % 8, (b - 2) % 4)

                @pl.when(t + 6 < CHT)
                def _():
                    edata_start(t + 6, (b + 6) % 8)

                @pl.when(t + 2 < CHT)
                def _():
                    edata_wait(t + 2, (b + 2) % 8)
                    gather_start(t + 2, (b + 2) % 8, (b + 2) % 4,
                                 hbm=(b % 2 == 1))
            return carry

        lax.fori_loop(0, CHT // 8, step, 0)
        scatter_wait(CHT - 2, (CHT - 2) % 8, (CHT - 2) % 4)
        scatter_wait(CHT - 1, (CHT - 1) % 8, (CHT - 1) % 4)

        plsc.subcore_barrier()
        pltpu.sync_copy(agg_sh.at[pl.ds(sid * RPT, RPT)],
                        agg_hbm.at[cid, pl.ds(sid * RPT, RPT)])

    return k


def _sc_edge_call(m2, edata, Hp2):
    return _sc_edge_kernel(Hp2)(m2, edata)


def _sc_score_call(g, srcc, dstc):
    cpt = TOTCH // 32

    @functools.partial(
        pl.kernel,
        out_type=jax.ShapeDtypeStruct((2, TOTCH, K), jnp.float32),
        mesh=_sc_mesh(),
        scratch_types=[
            pltpu.VMEM((NP, 4), jnp.float32),
            pltpu.VMEM((cpt, K), jnp.int32),
            pltpu.VMEM((cpt, K), jnp.int32),
            pltpu.VMEM((cpt, K), jnp.float32),
        ],
        compiler_params=_SC_PARAMS,
    )
    def k(g_hbm, src_hbm, dst_hbm, o_hbm, gv, src_v, dst_v, ov):
        cid = lax.axis_index("c")
        sid = lax.axis_index("s")
        wid = cid * 16 + sid
        pltpu.sync_copy(g_hbm, gv)
        pltpu.sync_copy(src_hbm.at[pl.ds(wid * cpt, cpt)], src_v)
        pltpu.sync_copy(dst_hbm.at[pl.ds(wid * cpt, cpt)], dst_v)

        for c in range(2):
            ca = jnp.full((16,), c, jnp.int32)
            cb = jnp.full((16,), 2 + c, jnp.int32)

            def chunk(t, carry):
                for grp in range(K // 16):
                    s16 = src_v[t, pl.ds(grp * 16, 16)]
                    d16 = dst_v[t, pl.ds(grp * 16, 16)]
                    av = plsc.load_gather(gv, [s16, ca])
                    bv = plsc.load_gather(gv, [d16, cb])
                    ov[t, pl.ds(grp * 16, 16)] = av + bv
                return carry

            lax.fori_loop(0, cpt, chunk, 0)
            pltpu.sync_copy(ov, o_hbm.at[c, pl.ds(wid * cpt, cpt)])

    return k(g, srcc, dstc)


def _rowblock_call(fn, blocked, full, outs):
    grid = (NP // BT,)
    in_specs = []
    for a in blocked:
        if a.ndim == 3:
            in_specs.append(
                pl.BlockSpec((2, BT, a.shape[2]), lambda i: (0, i, 0)))
        else:
            in_specs.append(
                pl.BlockSpec((BT, a.shape[1]), lambda i: (i, 0)))
    in_specs += [pl.BlockSpec(a.shape, lambda i, _n=a.ndim: (0,) * _n)
                 for a in full]
    out_specs, out_shape = [], []
    for o in outs:
        if isinstance(o, tuple):
            w = o[1]
            out_specs.append(pl.BlockSpec((2, BT, w), lambda i: (0, i, 0)))
            out_shape.append(jax.ShapeDtypeStruct((2, NP, w), jnp.float32))
        else:
            out_specs.append(pl.BlockSpec((BT, o), lambda i: (i, 0)))
            out_shape.append(jax.ShapeDtypeStruct((NP, o), jnp.float32))
    return pl.pallas_call(
        fn, grid=grid, in_specs=in_specs, out_specs=out_specs,
        out_shape=out_shape)(*blocked, *full)


def _dot(a, b):
    return jnp.dot(a, b, preferred_element_type=jnp.float32)


def _split_out(m, plan):
    parts, o = [], 0
    for w in plan:
        parts.append(jnp.stack([m[:, o:o + w], m[:, o + w:o + 2 * w]],
                               axis=0))
        o += 2 * w
    return parts


def _gru_fn(do_relu, Hp, Hpo, nagg, mplan):
    def f(x_r, *rest):
        agg_rs = rest[:nagg]
        Ar, Az, Ac, Br, Bz, Bc, bir, biz, bic, bhr, bhz, bhc = \
            rest[nagg:nagg + 12]
        rest = rest[nagg + 12:]
        if mplan:
            Wn_r = rest[0]
            ox_r = rest[1]
            om_rs = rest[2:]
        else:
            (ox_r,) = rest
        x = x_r[...]
        agg = jnp.concatenate(
            [a_r[c] for a_r in agg_rs for c in (0, 1)], axis=1)
        gr = _dot(agg, Ar[...]) + bir[...]
        gz = _dot(agg, Az[...]) + biz[...]
        gc = _dot(agg, Ac[...]) + bic[...]
        hr = _dot(x, Br[...]) + bhr[...]
        hz = _dot(x, Bz[...]) + bhz[...]
        hc = _dot(x, Bc[...]) + bhc[...]
        r = jax.nn.sigmoid(gr + hr)
        z = jax.nn.sigmoid(gz + hz)
        c = jnp.tanh(gc + r * hc)
        xn = (1.0 - z) * c + z * x
        if do_relu:
            xn = jnp.maximum(xn, 0.0)
        if Hpo > Hp:
            xn = jnp.concatenate(
                [xn, jnp.zeros((BT, Hpo - Hp), jnp.float32)], axis=1)
        ox_r[...] = xn
        if mplan:
            for om_r, part in zip(om_rs, _split_out(_dot(xn, Wn_r[...]),
                                                    mplan)):
                om_r[...] = part
    return f


def _embed_fn(x_r, W_r, om_r):
    (part,) = _split_out(_dot(x_r[...], W_r[...]), (32,))
    om_r[...] = part


def _final_fn(x_r, fcnW_r, fcnb_r, fe1a_r, fe1b_r, fe2W_r, fe1bias_r,
              fe2bias_r, g_r):
    h = jnp.maximum(_dot(x_r[...], fcnW_r[...]) + fcnb_r[...], 0.0)
    P1 = _dot(fe1a_r[...], fe2W_r[...])
    P2 = _dot(fe1b_r[...], fe2W_r[...])
    c0 = _dot(fe1bias_r[...], fe2W_r[...]) + fe2bias_r[...]
    g_r[...] = jnp.concatenate([_dot(h, P1), _dot(h, P2) + c0], axis=1)


def _prep_layer(W, wih, whh, bih, bhh, H, Hp):
    Wp = [jnp.pad(W[i], ((0, Hp - H), (0, Hp - H))) for i in range(3)]

    def split_t(w):
        return [jnp.pad(w[kk * H:(kk + 1) * H, :].T,
                        ((0, Hp - H), (0, Hp - H))) for kk in range(3)]

    A = split_t(wih)
    B = split_t(whh)
    bi = [jnp.pad(bih[kk * H:(kk + 1) * H], (0, Hp - H)).reshape(1, Hp)
          for kk in range(3)]
    bh = [jnp.pad(bhh[kk * H:(kk + 1) * H], (0, Hp - H)).reshape(1, Hp)
          for kk in range(3)]
    return Wp, A, B, bi, bh


def kernel(features, edge_index, distance,
           c1_W, c1_wih, c1_whh, c1_bih, c1_bhh,
           c2_W, c2_wih, c2_whh, c2_bih, c2_bhh,
           c3_W, c3_wih, c3_whh, c3_bih, c3_bhh,
           fcn_W, fcn_b, fe1_W, fe1_b, fe2_W, fe2_b):
    n = features.shape[0]
    e = distance.shape[0]
    pe = EP - e
    src = jnp.pad(edge_index[0], (0, pe)).reshape(TOTCH, K)
    dst = jnp.pad(edge_index[1], (0, pe)).reshape(TOTCH, K)
    ewb = lax.bitcast_convert_type(
        jnp.pad(distance, (0, pe)), jnp.int32).reshape(TOTCH, K)
    edata = jnp.stack([src, dst, ewb], axis=1)

    layers = [
        (_prep_layer(c1_W, c1_wih, c1_whh, c1_bih, c1_bhh, 50, 64), 64,
         (32,)),
        (_prep_layer(c2_W, c2_wih, c2_whh, c2_bih, c2_bhh, 100, 128), 128,
         (64,)),
        (_prep_layer(c3_W, c3_wih, c3_whh, c3_bih, c3_bhh, 150, 160), 160,
         (48, 32)),
    ]

    x = jnp.pad(features, ((0, NP - n), (0, 64 - features.shape[1])))
    m_parts = list(_rowblock_call(_embed_fn, [x], [layers[0][0][0][0]],
                                  [("split", 32)]))

    for li, ((Wp, A, B, bi, bh), Hp, plan) in enumerate(layers):
        for it in range(3):
            agg_parts = [_sc_edge_call(mp, edata, w)
                         for mp, w in zip(m_parts, plan)]
            last = (it == 2)
            if not last:
                Hpo, Wn, mplan = Hp, Wp[it + 1], plan
            elif li < 2:
                Hpo = layers[li + 1][1]
                Wn = layers[li + 1][0][0][0]
                mplan = layers[li + 1][2]
            else:
                Hpo, Wn, mplan = Hp, None, ()
            fn = _gru_fn(last, Hp, Hpo, len(agg_parts), mplan)
            blocked = [x] + agg_parts
            full = A + B + bi + bh
            outs = [Hpo]
            if mplan:
                full = full + [Wn]
                outs = [Hpo] + [("split", w) for w in mplan]
            res = _rowblock_call(fn, blocked, full, outs)
            x = res[0]
            m_parts = list(res[1:])

    fcnWp = jnp.pad(fcn_W, ((0, 10), (0, 10)))
    fcnbp = jnp.pad(fcn_b, (0, 10)).reshape(1, 160)
    fe1a = jnp.pad(fe1_W[:150], ((0, 10), (0, 0)))
    fe1b = jnp.pad(fe1_W[150:], ((0, 10), (0, 0)))
    (g,) = _rowblock_call(
        _final_fn, [x],
        [fcnWp, fcnbp, fe1a, fe1b, fe2_W, fe1_b.reshape(1, 100),
         fe2_b.reshape(1, 2)], [4])

    o = _sc_score_call(g, src, dst)
    out = jnp.stack([o[0].reshape(EP), o[1].reshape(EP)], axis=1)
    return out[:e]


# speedup vs baseline: 1.5161x; 1.5161x over previous
import functools

import jax
import jax.numpy as jnp
from jax import lax
from jax.experimental import pallas as pl
from jax.experimental.pallas import tpu as pltpu
from jax.experimental.pallas import tpu_sc as plsc

NP = 10240
K = 128
CHT = 160
TOTCH = 16 * CHT
EP = TOTCH * K
RPT = NP // 16
BT = 1024

_SC_PARAMS = pltpu.CompilerParams(needs_layout_passes=False,
                                  use_tc_tiling_on_sc=False)


def _sc_mesh():
    return plsc.VectorSubcoreMesh(core_axis_name="c", subcore_axis_name="s")


@functools.lru_cache(maxsize=None)
def _sc_edge_kernel(Hp2):
    nblk = Hp2 // 16

    @functools.partial(
        pl.kernel,
        out_type=jax.ShapeDtypeStruct((2, NP, Hp2), jnp.float32),
        mesh=_sc_mesh(),
        scratch_types=[
            pltpu.VMEM((8, 3, K), jnp.int32),
            pltpu.VMEM((4, K, Hp2), jnp.float32),
            pltpu.VMEM((64, Hp2), jnp.float32),
            pltpu.VMEM_SHARED((NP, Hp2), jnp.float32),
            pltpu.VMEM_SHARED((NP, Hp2), jnp.float32),
        ] + [pltpu.SemaphoreType.DMA] * 17,
        compiler_params=_SC_PARAMS,
    )
    def k(m_hbm, ed_hbm, agg_hbm, ebuf, rowsb, zbuf, agg_sh, m_sh, *sems):
        gsems = sems[0:4]
        ssems = sems[4:8]
        esems = sems[8:16]
        cid = lax.axis_index("c")
        sid = lax.axis_index("s")

        zeros16 = jnp.zeros((16,), jnp.float32)

        def zrow(i, carry):
            for j in range(nblk):
                zbuf[i, pl.ds(j * 16, 16)] = zeros16
            return carry

        lax.fori_loop(0, 64, zrow, 0)
        pltpu.async_copy(m_hbm.at[cid, pl.ds(sid * RPT, RPT)],
                         m_sh.at[pl.ds(sid * RPT, RPT)], sems[16])
        for r in range(RPT // 64):
            pltpu.async_copy(zbuf, agg_sh.at[pl.ds(sid * RPT + r * 64, 64)],
                             sems[16])
        pltpu.make_async_copy(m_hbm.at[cid, pl.ds(sid * RPT, RPT)],
                              m_sh.at[pl.ds(sid * RPT, RPT)],
                              sems[16]).wait()
        for r in range(RPT // 64):
            pltpu.make_async_copy(
                zbuf, agg_sh.at[pl.ds(sid * RPT + r * 64, 64)],
                sems[16]).wait()
        plsc.subcore_barrier()

        def edata_start(t, s):
            pltpu.async_copy(ed_hbm.at[sid * CHT + t], ebuf.at[s], esems[s])

        def edata_wait(t, s):
            pltpu.make_async_copy(ed_hbm.at[sid * CHT + t], ebuf.at[s],
                                  esems[s]).wait()

        def gather_start(t, s, r):
            pltpu.async_copy(m_sh.at[ebuf.at[s, 0]], rowsb.at[r],
                             gsems[r])

        def gather_wait(t, s, r):
            pltpu.make_async_copy(m_sh.at[ebuf.at[s, 0]],
                                  rowsb.at[r], gsems[r]).wait()

        def scatter_start(t, s, r):
            pltpu.async_copy(rowsb.at[r], agg_sh.at[ebuf.at[s, 1]], ssems[r],
                             add=True)

        def scatter_wait(t, s, r):
            pltpu.make_async_copy(rowsb.at[r], agg_sh.at[ebuf.at[s, 1]],
                                  ssems[r]).wait()

        def scale(s, r):
            s16 = jnp.full((16,), s, jnp.int32)
            two16 = jnp.full((16,), 2, jnp.int32)

            def ebody(g, carry):
                e0 = g * 4
                ews = []
                for u in range(4):
                    ewbits = plsc.load_gather(
                        ebuf, [s16, two16, jnp.full((16,), e0 + u,
                                                    jnp.int32)])
                    ews.append(plsc.bitcast(ewbits, jnp.float32))
                for u in range(4):
                    for j in range(nblk):
                        rowsb[r, e0 + u, pl.ds(j * 16, 16)] = (
                            rowsb[r, e0 + u, pl.ds(j * 16, 16)] * ews[u])
                return carry
            lax.fori_loop(0, K // 4, ebody, 0)

        for s in range(6):
            edata_start(s, s)
        edata_wait(0, 0)
        gather_start(0, 0, 0)
        edata_wait(1, 1)
        gather_start(1, 1, 1)

        def step(i, carry):
            for b in range(8):
                t = 8 * i + b
                gather_wait(t, b, b % 4)
                scale(b, b % 4)
                scatter_start(t, b, b % 4)

                @pl.when(t >= 2)
                def _():
                    scatter_wait(t - 2, (b - 2) % 8, (b - 2) % 4)

                @pl.when(t + 6 < CHT)
                def _():
                    edata_start(t + 6, (b + 6) % 8)

                @pl.when(t + 2 < CHT)
                def _():
                    edata_wait(t + 2, (b + 2) % 8)
                    gather_start(t + 2, (b + 2) % 8, (b + 2) % 4)
            return carry

        lax.fori_loop(0, CHT // 8, step, 0)
        scatter_wait(CHT - 2, (CHT - 2) % 8, (CHT - 2) % 4)
        scatter_wait(CHT - 1, (CHT - 1) % 8, (CHT - 1) % 4)

        plsc.subcore_barrier()
        pltpu.sync_copy(agg_sh.at[pl.ds(sid * RPT, RPT)],
                        agg_hbm.at[cid, pl.ds(sid * RPT, RPT)])

    return k


def _sc_edge_call(m2, edata, Hp2):
    return _sc_edge_kernel(Hp2)(m2, edata)


def _sc_score_call(g, srcc, dstc):
    cpt = TOTCH // 32

    @functools.partial(
        pl.kernel,
        out_type=jax.ShapeDtypeStruct((2, TOTCH, K), jnp.float32),
        mesh=_sc_mesh(),
        scratch_types=[
            pltpu.VMEM((NP, 4), jnp.float32),
            pltpu.VMEM((cpt, K), jnp.int32),
            pltpu.VMEM((cpt, K), jnp.int32),
            pltpu.VMEM((cpt, K), jnp.float32),
        ],
        compiler_params=_SC_PARAMS,
    )
    def k(g_hbm, src_hbm, dst_hbm, o_hbm, gv, src_v, dst_v, ov):
        cid = lax.axis_index("c")
        sid = lax.axis_index("s")
        wid = cid * 16 + sid
        pltpu.sync_copy(g_hbm, gv)
        pltpu.sync_copy(src_hbm.at[pl.ds(wid * cpt, cpt)], src_v)
        pltpu.sync_copy(dst_hbm.at[pl.ds(wid * cpt, cpt)], dst_v)

        for c in range(2):
            ca = jnp.full((16,), c, jnp.int32)
            cb = jnp.full((16,), 2 + c, jnp.int32)

            def chunk(t, carry):
                for grp in range(K // 16):
                    s16 = src_v[t, pl.ds(grp * 16, 16)]
                    d16 = dst_v[t, pl.ds(grp * 16, 16)]
                    av = plsc.load_gather(gv, [s16, ca])
                    bv = plsc.load_gather(gv, [d16, cb])
                    ov[t, pl.ds(grp * 16, 16)] = av + bv
                return carry

            lax.fori_loop(0, cpt, chunk, 0)
            pltpu.sync_copy(ov, o_hbm.at[c, pl.ds(wid * cpt, cpt)])

    return k(g, srcc, dstc)


def _rowblock_call(fn, blocked, full, outs):
    grid = (NP // BT,)
    in_specs = []
    for a in blocked:
        if a.ndim == 3:
            in_specs.append(
                pl.BlockSpec((2, BT, a.shape[2]), lambda i: (0, i, 0)))
        else:
            in_specs.append(
                pl.BlockSpec((BT, a.shape[1]), lambda i: (i, 0)))
    in_specs += [pl.BlockSpec(a.shape, lambda i, _n=a.ndim: (0,) * _n)
                 for a in full]
    out_specs, out_shape = [], []
    for o in outs:
        if isinstance(o, tuple):
            w = o[1]
            out_specs.append(pl.BlockSpec((2, BT, w), lambda i: (0, i, 0)))
            out_shape.append(jax.ShapeDtypeStruct((2, NP, w), jnp.float32))
        else:
            out_specs.append(pl.BlockSpec((BT, o), lambda i: (i, 0)))
            out_shape.append(jax.ShapeDtypeStruct((NP, o), jnp.float32))
    return pl.pallas_call(
        fn, grid=grid, in_specs=in_specs, out_specs=out_specs,
        out_shape=out_shape)(*blocked, *full)


def _dot(a, b):
    return jnp.dot(a, b, preferred_element_type=jnp.float32)


def _split_out(m, plan):
    parts, o = [], 0
    for w in plan:
        parts.append(jnp.stack([m[:, o:o + w], m[:, o + w:o + 2 * w]],
                               axis=0))
        o += 2 * w
    return parts


def _gru_fn(do_relu, Hp, Hpo, nagg, mplan):
    def f(x_r, *rest):
        agg_rs = rest[:nagg]
        Ar, Az, Ac, Br, Bz, Bc, bir, biz, bic, bhr, bhz, bhc = \
            rest[nagg:nagg + 12]
        rest = rest[nagg + 12:]
        if mplan:
            Wn_r = rest[0]
            ox_r = rest[1]
            om_rs = rest[2:]
        else:
            (ox_r,) = rest
        x = x_r[...]
        agg = jnp.concatenate(
            [a_r[c] for a_r in agg_rs for c in (0, 1)], axis=1)
        gr = _dot(agg, Ar[...]) + bir[...]
        gz = _dot(agg, Az[...]) + biz[...]
        gc = _dot(agg, Ac[...]) + bic[...]
        hr = _dot(x, Br[...]) + bhr[...]
        hz = _dot(x, Bz[...]) + bhz[...]
        hc = _dot(x, Bc[...]) + bhc[...]
        r = jax.nn.sigmoid(gr + hr)
        z = jax.nn.sigmoid(gz + hz)
        c = jnp.tanh(gc + r * hc)
        xn = (1.0 - z) * c + z * x
        if do_relu:
            xn = jnp.maximum(xn, 0.0)
        if Hpo > Hp:
            xn = jnp.concatenate(
                [xn, jnp.zeros((BT, Hpo - Hp), jnp.float32)], axis=1)
        ox_r[...] = xn
        if mplan:
            for om_r, part in zip(om_rs, _split_out(_dot(xn, Wn_r[...]),
                                                    mplan)):
                om_r[...] = part
    return f


def _embed_fn(x_r, W_r, om_r):
    (part,) = _split_out(_dot(x_r[...], W_r[...]), (32,))
    om_r[...] = part


def _final_fn(x_r, fcnW_r, fcnb_r, fe1a_r, fe1b_r, fe2W_r, fe1bias_r,
              fe2bias_r, g_r):
    h = jnp.maximum(_dot(x_r[...], fcnW_r[...]) + fcnb_r[...], 0.0)
    P1 = _dot(fe1a_r[...], fe2W_r[...])
    P2 = _dot(fe1b_r[...], fe2W_r[...])
    c0 = _dot(fe1bias_r[...], fe2W_r[...]) + fe2bias_r[...]
    g_r[...] = jnp.concatenate([_dot(h, P1), _dot(h, P2) + c0], axis=1)


def _prep_layer(W, wih, whh, bih, bhh, H, Hp):
    Wp = [jnp.pad(W[i], ((0, Hp - H), (0, Hp - H))) for i in range(3)]

    def split_t(w):
        return [jnp.pad(w[kk * H:(kk + 1) * H, :].T,
                        ((0, Hp - H), (0, Hp - H))) for kk in range(3)]

    A = split_t(wih)
    B = split_t(whh)
    bi = [jnp.pad(bih[kk * H:(kk + 1) * H], (0, Hp - H)).reshape(1, Hp)
          for kk in range(3)]
    bh = [jnp.pad(bhh[kk * H:(kk + 1) * H], (0, Hp - H)).reshape(1, Hp)
          for kk in range(3)]
    return Wp, A, B, bi, bh


def kernel(features, edge_index, distance,
           c1_W, c1_wih, c1_whh, c1_bih, c1_bhh,
           c2_W, c2_wih, c2_whh, c2_bih, c2_bhh,
           c3_W, c3_wih, c3_whh, c3_bih, c3_bhh,
           fcn_W, fcn_b, fe1_W, fe1_b, fe2_W, fe2_b):
    n = features.shape[0]
    e = distance.shape[0]
    pe = EP - e
    src = jnp.pad(edge_index[0], (0, pe)).reshape(TOTCH, K)
    dst = jnp.pad(edge_index[1], (0, pe)).reshape(TOTCH, K)
    ewb = lax.bitcast_convert_type(
        jnp.pad(distance, (0, pe)), jnp.int32).reshape(TOTCH, K)
    edata = jnp.stack([src, dst, ewb], axis=1)

    layers = [
        (_prep_layer(c1_W, c1_wih, c1_whh, c1_bih, c1_bhh, 50, 64), 64,
         (32,)),
        (_prep_layer(c2_W, c2_wih, c2_whh, c2_bih, c2_bhh, 100, 128), 128,
         (64,)),
        (_prep_layer(c3_W, c3_wih, c3_whh, c3_bih, c3_bhh, 150, 160), 160,
         (48, 32)),
    ]

    x = jnp.pad(features, ((0, NP - n), (0, 64 - features.shape[1])))
    m_parts = list(_rowblock_call(_embed_fn, [x], [layers[0][0][0][0]],
                                  [("split", 32)]))

    for li, ((Wp, A, B, bi, bh), Hp, plan) in enumerate(layers):
        for it in range(3):
            agg_parts = [_sc_edge_call(mp, edata, w)
                         for mp, w in zip(m_parts, plan)]
            last = (it == 2)
            if not last:
                Hpo, Wn, mplan = Hp, Wp[it + 1], plan
            elif li < 2:
                Hpo = layers[li + 1][1]
                Wn = layers[li + 1][0][0][0]
                mplan = layers[li + 1][2]
            else:
                Hpo, Wn, mplan = Hp, None, ()
            fn = _gru_fn(last, Hp, Hpo, len(agg_parts), mplan)
            blocked = [x] + agg_parts
            full = A + B + bi + bh
            outs = [Hpo]
            if mplan:
                full = full + [Wn]
                outs = [Hpo] + [("split", w) for w in mplan]
            res = _rowblock_call(fn, blocked, full, outs)
            x = res[0]
            m_parts = list(res[1:])

    fcnWp = jnp.pad(fcn_W, ((0, 10), (0, 10)))
    fcnbp = jnp.pad(fcn_b, (0, 10)).reshape(1, 160)
    fe1a = jnp.pad(fe1_W[:150], ((0, 10), (0, 0)))
    fe1b = jnp.pad(fe1_W[150:], ((0, 10), (0, 0)))
    (g,) = _rowblock_call(
        _final_fn, [x],
        [fcnWp, fcnbp, fe1a, fe1b, fe2_W, fe1_b.reshape(1, 100),
         fe2_b.reshape(1, 2)], [4])

    o = _sc_score_call(g, src, dst)
    out = jnp.stack([o[0].reshape(EP), o[1].reshape(EP)], axis=1)
    return out[:e]
